# baseline (device time: 746139 ns/iter reference)
import jax
import jax.numpy as jnp
from jax import lax
from jax.experimental import pallas as pl
from jax.experimental.pallas import tpu as pltpu

N_DEV = 16
N_SLOTS = 4


def kernel(x, w_mat):
    x = x.astype(jnp.bfloat16)
    w = w_mat.astype(jnp.bfloat16)
    m, _ = x.shape
    n = w.shape[1]
    chunk_m = m // N_DEV

    def body(x_ref, w_ref, out_ref, comm_ref, send_sems, recv_sems):
        d = lax.axis_index("i")
        right = (d + 1) % N_DEV
        left = (d - 1) % N_DEV

        barrier_sem = pltpu.get_barrier_semaphore()
        for nbr in (left, right):
            pl.semaphore_signal(
                barrier_sem, inc=1,
                device_id=(nbr,), device_id_type=pl.DeviceIdType.MESH,
            )
        pl.semaphore_wait(barrier_sem, 2)

        def partial_f32(c):
            rows = x_ref[pl.ds(c * chunk_m, chunk_m), :]
            return lax.dot_general(
                rows, w_ref[:, :],
                dimension_numbers=(((1,), (0,)), ((), ())),
                preferred_element_type=jnp.float32,
            )

        c0 = (d - 1) % N_DEV
        comm_ref[0, :, :] = partial_f32(c0).astype(jnp.bfloat16)

        for s in range(N_DEV - 1):
            send_slot = s % N_SLOTS
            recv_slot = (s + 1) % N_SLOTS
            rdma = pltpu.make_async_remote_copy(
                src_ref=comm_ref.at[send_slot],
                dst_ref=comm_ref.at[recv_slot],
                send_sem=send_sems.at[send_slot],
                recv_sem=recv_sems.at[recv_slot],
                device_id=(right,),
                device_id_type=pl.DeviceIdType.MESH,
            )
            rdma.start()
            p = partial_f32((d - 2 - s) % N_DEV)
            rdma.wait()
            if s < N_DEV - 2:
                comm_ref[recv_slot, :, :] = (
                    comm_ref[recv_slot, :, :].astype(jnp.float32) + p
                ).astype(jnp.bfloat16)
            else:
                y = comm_ref[recv_slot, :, :].astype(jnp.float32) + p
                out_ref[:, :] = y * jax.nn.sigmoid(y)

    return pl.pallas_call(
        body,
        out_shape=jax.ShapeDtypeStruct((chunk_m, n), jnp.float32),
        in_specs=[
            pl.BlockSpec(memory_space=pltpu.VMEM),
            pl.BlockSpec(memory_space=pltpu.VMEM),
        ],
        out_specs=pl.BlockSpec(memory_space=pltpu.VMEM),
        scratch_shapes=[
            pltpu.VMEM((N_SLOTS, chunk_m, n), jnp.bfloat16),
            pltpu.SemaphoreType.DMA((N_SLOTS,)),
            pltpu.SemaphoreType.DMA((N_SLOTS,)),
        ],
        compiler_params=pltpu.CompilerParams(collective_id=0),
    )(x, w)


# device time: 416379 ns/iter; 1.7920x vs baseline; 1.7920x over previous
import jax
import jax.numpy as jnp
from jax import lax
from jax.experimental import pallas as pl
from jax.experimental.pallas import tpu as pltpu

N_DEV = 16
N_SLOTS = 4


def kernel(x, w_mat):
    x = x.astype(jnp.bfloat16)
    w = w_mat.astype(jnp.bfloat16)
    m, _ = x.shape
    n = w.shape[1]
    chunk_m = m // N_DEV
    nh = n // 2

    def body(x_ref, w_ref, out_ref, comm_r, comm_l,
             send_r, recv_r, send_l, recv_l):
        d = lax.axis_index("i")
        right = (d + 1) % N_DEV
        left = (d - 1) % N_DEV

        barrier_sem = pltpu.get_barrier_semaphore()
        for nbr in (left, right):
            pl.semaphore_signal(
                barrier_sem, inc=1,
                device_id=(nbr,), device_id_type=pl.DeviceIdType.MESH,
            )
        pl.semaphore_wait(barrier_sem, 2)

        def partial_f32(c, col0):
            rows = x_ref[pl.ds(c * chunk_m, chunk_m), :]
            return lax.dot_general(
                rows, w_ref[:, col0:col0 + nh],
                dimension_numbers=(((1,), (0,)), ((), ())),
                preferred_element_type=jnp.float32,
            )

        comm_r[0, :, :] = partial_f32((d - 1) % N_DEV, 0).astype(jnp.bfloat16)
        comm_l[0, :, :] = partial_f32((d + 1) % N_DEV, nh).astype(jnp.bfloat16)

        def start_hop(s, send_slot, recv_slot):
            rdma_r = pltpu.make_async_remote_copy(
                src_ref=comm_r.at[send_slot],
                dst_ref=comm_r.at[recv_slot],
                send_sem=send_r.at[send_slot],
                recv_sem=recv_r.at[recv_slot],
                device_id=(right,),
                device_id_type=pl.DeviceIdType.MESH,
            )
            rdma_l = pltpu.make_async_remote_copy(
                src_ref=comm_l.at[send_slot],
                dst_ref=comm_l.at[recv_slot],
                send_sem=send_l.at[send_slot],
                recv_sem=recv_l.at[recv_slot],
                device_id=(left,),
                device_id_type=pl.DeviceIdType.MESH,
            )
            rdma_r.start()
            rdma_l.start()
            p_r = partial_f32((d - 2 - s) % N_DEV, 0)
            p_l = partial_f32((d + 2 + s) % N_DEV, nh)
            rdma_r.wait()
            rdma_l.wait()
            return p_r, p_l

        def hop(s, _):
            send_slot = s % N_SLOTS
            recv_slot = (s + 1) % N_SLOTS
            p_r, p_l = start_hop(s, send_slot, recv_slot)
            comm_r[recv_slot, :, :] = (
                comm_r[recv_slot, :, :].astype(jnp.float32) + p_r
            ).astype(jnp.bfloat16)
            comm_l[recv_slot, :, :] = (
                comm_l[recv_slot, :, :].astype(jnp.float32) + p_l
            ).astype(jnp.bfloat16)
            return 0

        lax.fori_loop(0, N_DEV - 2, hop, 0)

        s_last = N_DEV - 2
        p_r, p_l = start_hop(s_last, s_last % N_SLOTS, (s_last + 1) % N_SLOTS)
        recv_slot = (s_last + 1) % N_SLOTS
        y_r = comm_r[recv_slot, :, :].astype(jnp.float32) + p_r
        out_ref[:, 0:nh] = y_r * jax.nn.sigmoid(y_r)
        y_l = comm_l[recv_slot, :, :].astype(jnp.float32) + p_l
        out_ref[:, nh:n] = y_l * jax.nn.sigmoid(y_l)

    return pl.pallas_call(
        body,
        out_shape=jax.ShapeDtypeStruct((chunk_m, n), jnp.float32),
        in_specs=[
            pl.BlockSpec(memory_space=pltpu.VMEM),
            pl.BlockSpec(memory_space=pltpu.VMEM),
        ],
        out_specs=pl.BlockSpec(memory_space=pltpu.VMEM),
        scratch_shapes=[
            pltpu.VMEM((N_SLOTS, chunk_m, nh), jnp.bfloat16),
            pltpu.VMEM((N_SLOTS, chunk_m, nh), jnp.bfloat16),
            pltpu.SemaphoreType.DMA((N_SLOTS,)),
            pltpu.SemaphoreType.DMA((N_SLOTS,)),
            pltpu.SemaphoreType.DMA((N_SLOTS,)),
            pltpu.SemaphoreType.DMA((N_SLOTS,)),
        ],
        compiler_params=pltpu.CompilerParams(collective_id=0),
    )(x, w)


# device time: 361942 ns/iter; 2.0615x vs baseline; 1.1504x over previous
import jax
import jax.numpy as jnp
from jax import lax
from jax.experimental import pallas as pl
from jax.experimental.pallas import tpu as pltpu

N_DEV = 16
N_SLOTS = 4


def kernel(x, w_mat):
    x = x.astype(jnp.bfloat16)
    w = w_mat.astype(jnp.bfloat16)
    m, _ = x.shape
    n = w.shape[1]
    chunk_m = m // N_DEV
    nh = n // 2
    nl = n // 4

    def body(x_ref, w_ref, out_ref,
             buf_r0, buf_r1, buf_l0, buf_l1,
             ss_r0, rs_r0, ss_r1, rs_r1,
             ss_l0, rs_l0, ss_l1, rs_l1):
        d = lax.axis_index("i")
        right = (d + 1) % N_DEV
        left = (d - 1) % N_DEV

        barrier_sem = pltpu.get_barrier_semaphore()
        for nbr in (left, right):
            pl.semaphore_signal(
                barrier_sem, inc=1,
                device_id=(nbr,), device_id_type=pl.DeviceIdType.MESH,
            )
        pl.semaphore_wait(barrier_sem, 2)

        def partial_f32(c, col0):
            rows = x_ref[pl.ds(c * chunk_m, chunk_m), :]
            return lax.dot_general(
                rows, w_ref[:, col0:col0 + nh],
                dimension_numbers=(((1,), (0,)), ((), ())),
                preferred_element_type=jnp.float32,
            )

        r_streams = ((buf_r0, ss_r0, rs_r0, 0), (buf_r1, ss_r1, rs_r1, nl))
        l_streams = ((buf_l0, ss_l0, rs_l0, 0), (buf_l1, ss_l1, rs_l1, nl))

        def rdma(buf, ss, rs, s_slot, r_slot, dev):
            return pltpu.make_async_remote_copy(
                src_ref=buf.at[s_slot],
                dst_ref=buf.at[r_slot],
                send_sem=ss.at[s_slot],
                recv_sem=rs.at[r_slot],
                device_id=(dev,),
                device_id_type=pl.DeviceIdType.MESH,
            )

        p_r = partial_f32((d - 1) % N_DEV, 0)
        p_l = partial_f32((d + 1) % N_DEV, nh)
        for (buf, ss, rs, c0), p, dev in (
            (r_streams[0], p_r, right), (r_streams[1], p_r, right),
            (l_streams[0], p_l, left), (l_streams[1], p_l, left),
        ):
            buf[0, :, :] = p[:, c0:c0 + nl].astype(jnp.bfloat16)
            rdma(buf, ss, rs, 0, 1, dev).start()

        def hop(s, _):
            sl_s = s % N_SLOTS
            sl_r = (s + 1) % N_SLOTS
            sl_n = (s + 2) % N_SLOTS
            p_r = partial_f32((d - 2 - s) % N_DEV, 0)
            p_l = partial_f32((d + 2 + s) % N_DEV, nh)
            for streams, p, dev in ((r_streams, p_r, right),
                                    (l_streams, p_l, left)):
                for buf, ss, rs, c0 in streams:
                    rdma(buf, ss, rs, sl_s, sl_r, dev).wait()
                    buf[sl_r, :, :] = (
                        buf[sl_r, :, :].astype(jnp.float32)
                        + p[:, c0:c0 + nl]
                    ).astype(jnp.bfloat16)
                    rdma(buf, ss, rs, sl_r, sl_n, dev).start()
            return 0

        lax.fori_loop(0, N_DEV - 2, hop, 0)

        s_last = N_DEV - 2
        sl_s = s_last % N_SLOTS
        sl_r = (s_last + 1) % N_SLOTS
        p_r = partial_f32(d, 0)
        p_l = partial_f32(d, nh)
        for streams, p, dev, out0 in ((r_streams, p_r, right, 0),
                                      (l_streams, p_l, left, nh)):
            for buf, ss, rs, c0 in streams:
                rdma(buf, ss, rs, sl_s, sl_r, dev).wait()
                y = buf[sl_r, :, :].astype(jnp.float32) + p[:, c0:c0 + nl]
                out_ref[:, out0 + c0:out0 + c0 + nl] = y * jax.nn.sigmoid(y)

    return pl.pallas_call(
        body,
        out_shape=jax.ShapeDtypeStruct((chunk_m, n), jnp.float32),
        in_specs=[
            pl.BlockSpec(memory_space=pltpu.VMEM),
            pl.BlockSpec(memory_space=pltpu.VMEM),
        ],
        out_specs=pl.BlockSpec(memory_space=pltpu.VMEM),
        scratch_shapes=(
            [pltpu.VMEM((N_SLOTS, chunk_m, nl), jnp.bfloat16)] * 4
            + [pltpu.SemaphoreType.DMA((N_SLOTS,))] * 8
        ),
        compiler_params=pltpu.CompilerParams(collective_id=0),
    )(x, w)
